# baseline (speedup 1.0000x reference)
import functools
import math

import jax
import jax.numpy as jnp
from jax import lax
from jax.experimental import pallas as pl

B, S, D = 1, 2048, 768
H, E, K, P = 12, 8, 2, 64
HP = H * P
HEp = 128
EHP = E * H * P
SBLK = 256
NBLK = S // SBLK

_NEG = -1e30


def _rot_lanes(x, s):
    n = x.shape[-1]
    s = s % n
    if s == 0:
        return x
    return jnp.concatenate([x[:, s:], x[:, :s]], axis=1)


def _rot_group8(x, s, e_idx):
    a = _rot_lanes(x, s)
    b = _rot_lanes(x, s - 8)
    return jnp.where(e_idx < 8 - s, a, b)


def _group8_reduce(x, e_idx, op):
    for s in (4, 2, 1):
        x = op(x, _rot_group8(x, s, e_idx))
    return x


def _top2_gate(probs, e_idx):
    fmax = jnp.maximum
    imin = jnp.minimum
    m1 = _group8_reduce(probs, e_idx, fmax)
    cand1 = jnp.where(probs == m1, e_idx, 8)
    i1 = _group8_reduce(cand1, e_idx, imin)
    probs2 = jnp.where(e_idx == i1, jnp.full_like(probs, _NEG), probs)
    m2 = _group8_reduce(probs2, e_idx, fmax)
    cand2 = jnp.where(probs2 == m2, e_idx, 8)
    i2 = _group8_reduce(cand2, e_idx, imin)
    denom = fmax(m1 + m2, 1e-9)
    gate = jnp.where(e_idx == i1, m1, jnp.where(e_idx == i2, m2, 0.0))
    return gate / denom


def _proj_kernel(x_ref, bigw_ref, selw_ref, rep_ref, q_ref, k_ref, vmix_ref,
                 go_ref):
    xb = x_ref[...]
    y = jnp.dot(xb.astype(jnp.bfloat16), bigw_ref[...],
                preferred_element_type=jnp.float32)
    q_ref[...] = y[:, :HP].astype(jnp.bfloat16)
    k_ref[...] = y[:, HP:2 * HP].astype(jnp.bfloat16)
    logits = jnp.dot(xb, selw_ref[...], preferred_element_type=jnp.float32)
    e_idx = lax.broadcasted_iota(jnp.int32, (SBLK, HEp), 1) % 8
    probs_v = jax.nn.sigmoid(logits[:, :HEp])
    probs_o = jax.nn.sigmoid(logits[:, HEp:])
    gate_v = _top2_gate(probs_v, e_idx)
    go_ref[...] = _top2_gate(probs_o, e_idx)
    allv = y[:, 2 * HP:]
    gate_big = jnp.dot(gate_v, rep_ref[...], preferred_element_type=jnp.float32)
    prod = allv * gate_big
    acc = prod[:, :HP]
    for e in range(1, E):
        acc = acc + prod[:, e * HP:(e + 1) * HP]
    vmix_ref[...] = acc.astype(jnp.bfloat16)


def _attn_out_kernel(q_ref, k_ref, v_ref, go_ref, rep_ref, o2_ref, out_ref):
    qi = pl.program_id(0)
    row = qi * SBLK + lax.broadcasted_iota(jnp.int32, (SBLK, SBLK), 0)
    col0 = lax.broadcasted_iota(jnp.int32, (SBLK, SBLK), 1)
    parts = []
    for h in range(H):
        sl = slice(h * P, (h + 1) * P)
        qh = q_ref[:, sl]

        def body(j, carry, qh=qh, sl=sl):
            m, l, acc = carry
            kh = k_ref[pl.ds(j * SBLK, SBLK), sl]
            sc = lax.dot_general(qh, kh, (((1,), (1,)), ((), ())),
                                 preferred_element_type=jnp.float32)
            sc = jnp.where(j * SBLK + col0 <= row, sc, _NEG)
            mn = jnp.maximum(m, jnp.max(sc, axis=1, keepdims=True))
            p = jnp.exp(sc - mn)
            corr = jnp.exp(m - mn)
            vh = v_ref[pl.ds(j * SBLK, SBLK), sl]
            acc = acc * corr + jnp.dot(p.astype(jnp.bfloat16), vh,
                                       preferred_element_type=jnp.float32)
            l = l * corr + jnp.sum(p, axis=1, keepdims=True)
            return mn, l, acc

        m0 = jnp.full((SBLK, 1), _NEG, jnp.float32)
        l0 = jnp.zeros((SBLK, 1), jnp.float32)
        a0 = jnp.zeros((SBLK, P), jnp.float32)
        m, l, acc = lax.fori_loop(0, qi + 1, body, (m0, l0, a0))
        parts.append(acc / l)
    res = jnp.concatenate(parts, axis=1)
    gate_big = jnp.dot(go_ref[...], rep_ref[...],
                       preferred_element_type=jnp.float32)
    res8 = jnp.concatenate([res] * E, axis=1)
    out_ref[...] = jnp.dot((res8 * gate_big).astype(jnp.bfloat16), o2_ref[...],
                           preferred_element_type=jnp.float32)


def kernel(x, Wq, Wk, v, o, sel_v, sel_o, route_scale):
    s = float(P) ** -0.25
    xf = x[0]
    pad = jnp.zeros((D, HEp - H * E), jnp.float32)
    bigw = jnp.concatenate([
        Wq.T * s, Wk.T * s,
        v.reshape(H, E, D, P).transpose(2, 1, 0, 3).reshape(D, EHP),
    ], axis=1).astype(jnp.bfloat16)
    selw = jnp.concatenate([sel_v.T, pad, sel_o.T, pad], axis=1)

    r = jnp.arange(HEp)[:, None]
    c = jnp.arange(EHP)[None, :]
    rep = (((r % 8) == (c // HP)) & ((r // 8) == ((c % HP) // P)) & (r < H * E))
    rep = rep.astype(jnp.float32) * route_scale[0]

    q2, k2, vmix2, gate_o = pl.pallas_call(
        _proj_kernel,
        grid=(NBLK,),
        in_specs=[
            pl.BlockSpec((SBLK, D), lambda i: (i, 0)),
            pl.BlockSpec((D, 2 * HP + EHP), lambda i: (0, 0)),
            pl.BlockSpec((D, 2 * HEp), lambda i: (0, 0)),
            pl.BlockSpec((HEp, EHP), lambda i: (0, 0)),
        ],
        out_specs=[
            pl.BlockSpec((SBLK, HP), lambda i: (i, 0)),
            pl.BlockSpec((SBLK, HP), lambda i: (i, 0)),
            pl.BlockSpec((SBLK, HP), lambda i: (i, 0)),
            pl.BlockSpec((SBLK, HEp), lambda i: (i, 0)),
        ],
        out_shape=[
            jax.ShapeDtypeStruct((S, HP), jnp.bfloat16),
            jax.ShapeDtypeStruct((S, HP), jnp.bfloat16),
            jax.ShapeDtypeStruct((S, HP), jnp.bfloat16),
            jax.ShapeDtypeStruct((S, HEp), jnp.float32),
        ],
    )(xf, bigw, selw, rep)

    o2e = o.reshape(H, E, P, D).transpose(1, 0, 2, 3).reshape(EHP, D)
    o2e = o2e.astype(jnp.bfloat16)

    out = pl.pallas_call(
        _attn_out_kernel,
        grid=(NBLK,),
        in_specs=[
            pl.BlockSpec((SBLK, HP), lambda i: (i, 0)),
            pl.BlockSpec((S, HP), lambda i: (0, 0)),
            pl.BlockSpec((S, HP), lambda i: (0, 0)),
            pl.BlockSpec((SBLK, HEp), lambda i: (i, 0)),
            pl.BlockSpec((HEp, EHP), lambda i: (0, 0)),
            pl.BlockSpec((EHP, D), lambda i: (0, 0)),
        ],
        out_specs=pl.BlockSpec((SBLK, D), lambda i: (i, 0)),
        out_shape=jax.ShapeDtypeStruct((S, D), jnp.float32),
    )(q2, k2, vmix2, gate_o, rep, o2e)

    return out.reshape(B, S, D)


# speedup vs baseline: 1.4499x; 1.4499x over previous
import functools
import math

import jax
import jax.numpy as jnp
from jax import lax
from jax.experimental import pallas as pl

B, S, D = 1, 2048, 768
H, E, K, P = 12, 8, 2, 64
HP = H * P
HEp = 128
EHP = E * H * P
SBLK = 256
NBLK = S // SBLK

_NEG = -1e30


def _rot_lanes(x, s):
    n = x.shape[-1]
    s = s % n
    if s == 0:
        return x
    return jnp.concatenate([x[:, s:], x[:, :s]], axis=1)


def _rot_group8(x, s, e_idx):
    a = _rot_lanes(x, s)
    b = _rot_lanes(x, s - 8)
    return jnp.where(e_idx < 8 - s, a, b)


def _group8_reduce(x, e_idx, op):
    for s in (4, 2, 1):
        x = op(x, _rot_group8(x, s, e_idx))
    return x


def _top2_gate(probs, e_idx):
    fmax = jnp.maximum
    imin = jnp.minimum
    m1 = _group8_reduce(probs, e_idx, fmax)
    cand1 = jnp.where(probs == m1, e_idx, 8)
    i1 = _group8_reduce(cand1, e_idx, imin)
    probs2 = jnp.where(e_idx == i1, jnp.full_like(probs, _NEG), probs)
    m2 = _group8_reduce(probs2, e_idx, fmax)
    cand2 = jnp.where(probs2 == m2, e_idx, 8)
    i2 = _group8_reduce(cand2, e_idx, imin)
    denom = fmax(m1 + m2, 1e-9)
    gate = jnp.where(e_idx == i1, m1, jnp.where(e_idx == i2, m2, 0.0))
    return gate / denom


def _proj_kernel(x_ref, bigw_ref, selw_ref, rep_ref, q_ref, k_ref, vmix_ref,
                 go_ref):
    xb = x_ref[...]
    y = jnp.dot(xb.astype(jnp.bfloat16), bigw_ref[...],
                preferred_element_type=jnp.float32)
    q_ref[...] = y[:, :HP].astype(jnp.bfloat16)
    k_ref[...] = y[:, HP:2 * HP].astype(jnp.bfloat16)
    logits = jnp.dot(xb, selw_ref[...], preferred_element_type=jnp.float32)
    e_idx = lax.broadcasted_iota(jnp.int32, (SBLK, HEp), 1) % 8
    probs_v = jax.nn.sigmoid(logits[:, :HEp])
    probs_o = jax.nn.sigmoid(logits[:, HEp:])
    gate_v = _top2_gate(probs_v, e_idx)
    go_ref[...] = _top2_gate(probs_o, e_idx)
    allv = y[:, 2 * HP:]
    gate_big = jnp.dot(gate_v, rep_ref[...], preferred_element_type=jnp.float32)
    prod = allv * gate_big
    acc = prod[:, :HP]
    for e in range(1, E):
        acc = acc + prod[:, e * HP:(e + 1) * HP]
    vmix_ref[...] = acc.astype(jnp.bfloat16)


def _attn_out_kernel(q_ref, k_ref, v_ref, go_ref, rep_ref, o2_ref, out_ref):
    qi = pl.program_id(0)
    row = qi * SBLK + lax.broadcasted_iota(jnp.int32, (SBLK, S), 0)
    col = lax.broadcasted_iota(jnp.int32, (SBLK, S), 1)
    causal = col <= row
    parts = []
    for h in range(H):
        sl = slice(h * P, (h + 1) * P)
        scores = lax.dot_general(q_ref[:, sl], k_ref[:, sl],
                                 (((1,), (1,)), ((), ())),
                                 preferred_element_type=jnp.float32)
        scores = jnp.where(causal, scores, _NEG)
        m = jnp.max(scores, axis=1, keepdims=True)
        p = jnp.exp(scores - m)
        denom = jnp.sum(p, axis=1, keepdims=True)
        acc = jnp.dot(p.astype(jnp.bfloat16), v_ref[:, sl],
                      preferred_element_type=jnp.float32)
        parts.append(acc / denom)
    res = jnp.concatenate(parts, axis=1)
    gate_big = jnp.dot(go_ref[...], rep_ref[...],
                       preferred_element_type=jnp.float32)
    res8 = jnp.concatenate([res] * E, axis=1)
    out_ref[...] = jnp.dot((res8 * gate_big).astype(jnp.bfloat16), o2_ref[...],
                           preferred_element_type=jnp.float32)


def kernel(x, Wq, Wk, v, o, sel_v, sel_o, route_scale):
    s = float(P) ** -0.25
    xf = x[0]
    pad = jnp.zeros((D, HEp - H * E), jnp.float32)
    bigw = jnp.concatenate([
        Wq.T * s, Wk.T * s,
        v.reshape(H, E, D, P).transpose(2, 1, 0, 3).reshape(D, EHP),
    ], axis=1).astype(jnp.bfloat16)
    selw = jnp.concatenate([sel_v.T, pad, sel_o.T, pad], axis=1)

    r = jnp.arange(HEp)[:, None]
    c = jnp.arange(EHP)[None, :]
    rep = (((r % 8) == (c // HP)) & ((r // 8) == ((c % HP) // P)) & (r < H * E))
    rep = rep.astype(jnp.float32) * route_scale[0]

    q2, k2, vmix2, gate_o = pl.pallas_call(
        _proj_kernel,
        grid=(NBLK,),
        in_specs=[
            pl.BlockSpec((SBLK, D), lambda i: (i, 0)),
            pl.BlockSpec((D, 2 * HP + EHP), lambda i: (0, 0)),
            pl.BlockSpec((D, 2 * HEp), lambda i: (0, 0)),
            pl.BlockSpec((HEp, EHP), lambda i: (0, 0)),
        ],
        out_specs=[
            pl.BlockSpec((SBLK, HP), lambda i: (i, 0)),
            pl.BlockSpec((SBLK, HP), lambda i: (i, 0)),
            pl.BlockSpec((SBLK, HP), lambda i: (i, 0)),
            pl.BlockSpec((SBLK, HEp), lambda i: (i, 0)),
        ],
        out_shape=[
            jax.ShapeDtypeStruct((S, HP), jnp.bfloat16),
            jax.ShapeDtypeStruct((S, HP), jnp.bfloat16),
            jax.ShapeDtypeStruct((S, HP), jnp.bfloat16),
            jax.ShapeDtypeStruct((S, HEp), jnp.float32),
        ],
    )(xf, bigw, selw, rep)

    o2e = o.reshape(H, E, P, D).transpose(1, 0, 2, 3).reshape(EHP, D)
    o2e = o2e.astype(jnp.bfloat16)

    out = pl.pallas_call(
        _attn_out_kernel,
        grid=(NBLK,),
        in_specs=[
            pl.BlockSpec((SBLK, HP), lambda i: (i, 0)),
            pl.BlockSpec((S, HP), lambda i: (0, 0)),
            pl.BlockSpec((S, HP), lambda i: (0, 0)),
            pl.BlockSpec((SBLK, HEp), lambda i: (i, 0)),
            pl.BlockSpec((HEp, EHP), lambda i: (0, 0)),
            pl.BlockSpec((EHP, D), lambda i: (0, 0)),
        ],
        out_specs=pl.BlockSpec((SBLK, D), lambda i: (i, 0)),
        out_shape=jax.ShapeDtypeStruct((S, D), jnp.float32),
    )(q2, k2, vmix2, gate_o, rep, o2e)

    return out.reshape(B, S, D)


# speedup vs baseline: 1.6999x; 1.1725x over previous
import functools
import math

import jax
import jax.numpy as jnp
from jax import lax
from jax.experimental import pallas as pl

B, S, D = 1, 2048, 768
H, E, K, P = 12, 8, 2, 64
HP = H * P
HEp = 128
EHP = E * H * P
SBLK = 256
NBLK = S // SBLK

_NEG = -1e30


def _rot_lanes(x, s):
    n = x.shape[-1]
    s = s % n
    if s == 0:
        return x
    return jnp.concatenate([x[:, s:], x[:, :s]], axis=1)


def _rot_group8(x, s, e_idx):
    a = _rot_lanes(x, s)
    b = _rot_lanes(x, s - 8)
    return jnp.where(e_idx < 8 - s, a, b)


def _group8_reduce(x, e_idx, op):
    for s in (4, 2, 1):
        x = op(x, _rot_group8(x, s, e_idx))
    return x


def _top2_gate(probs, e_idx):
    fmax = jnp.maximum
    imin = jnp.minimum
    m1 = _group8_reduce(probs, e_idx, fmax)
    cand1 = jnp.where(probs == m1, e_idx, 8)
    i1 = _group8_reduce(cand1, e_idx, imin)
    probs2 = jnp.where(e_idx == i1, jnp.full_like(probs, _NEG), probs)
    m2 = _group8_reduce(probs2, e_idx, fmax)
    cand2 = jnp.where(probs2 == m2, e_idx, 8)
    i2 = _group8_reduce(cand2, e_idx, imin)
    denom = fmax(m1 + m2, 1e-9)
    gate = jnp.where(e_idx == i1, m1, jnp.where(e_idx == i2, m2, 0.0))
    return gate / denom


def _proj_kernel(x_ref, wqk_ref, v2e_ref, selw_ref, rep_ref, q_ref, k_ref,
                 vmix_ref, go_ref):
    xb = x_ref[...]
    x16 = xb.astype(jnp.bfloat16)
    qk = jnp.dot(x16, wqk_ref[...], preferred_element_type=jnp.float32)
    q_ref[...] = qk[:, :HP].astype(jnp.bfloat16)
    k_ref[...] = qk[:, HP:].astype(jnp.bfloat16)
    logits = jnp.dot(xb, selw_ref[...], preferred_element_type=jnp.float32)
    e_idx = lax.broadcasted_iota(jnp.int32, (SBLK, HEp), 1) % 8
    probs_v = jax.nn.sigmoid(logits[:, :HEp])
    probs_o = jax.nn.sigmoid(logits[:, HEp:])
    gate_v = _top2_gate(probs_v, e_idx)
    go_ref[...] = _top2_gate(probs_o, e_idx)
    allv = jnp.dot(x16, v2e_ref[...], preferred_element_type=jnp.float32)
    gate_big = jnp.dot(gate_v.astype(jnp.bfloat16), rep_ref[...],
                       preferred_element_type=jnp.float32)
    prod = allv * gate_big
    acc = prod[:, :HP]
    for e in range(1, E):
        acc = acc + prod[:, e * HP:(e + 1) * HP]
    vmix_ref[...] = acc.astype(jnp.bfloat16)


def _attn_kernel(q_ref, k_ref, v_ref, o_ref, q_off=0):
    qi = pl.program_id(0) + q_off
    skv = k_ref.shape[0]
    row = qi * SBLK + lax.broadcasted_iota(jnp.int32, (SBLK, skv), 0)
    col = lax.broadcasted_iota(jnp.int32, (SBLK, skv), 1)
    causal = col <= row
    for h in range(H):
        sl = slice(h * P, (h + 1) * P)
        scores = lax.dot_general(q_ref[:, sl], k_ref[:, sl],
                                 (((1,), (1,)), ((), ())),
                                 preferred_element_type=jnp.float32)
        scores = jnp.where(causal, scores, _NEG)
        m = jnp.max(scores, axis=1, keepdims=True)
        p = jnp.exp(scores - m)
        denom = jnp.sum(p, axis=1, keepdims=True)
        acc = jnp.dot(p.astype(jnp.bfloat16), v_ref[:, sl],
                      preferred_element_type=jnp.float32)
        o_ref[:, sl] = acc / denom


def _out_kernel(res_ref, go_ref, rep_ref, o2_ref, out_ref):
    res = res_ref[...]
    gate_big = jnp.dot(go_ref[...].astype(jnp.bfloat16), rep_ref[...],
                       preferred_element_type=jnp.float32)
    res8 = jnp.concatenate([res] * E, axis=1)
    out_ref[...] = jnp.dot((res8 * gate_big).astype(jnp.bfloat16), o2_ref[...],
                           preferred_element_type=jnp.float32)


def kernel(x, Wq, Wk, v, o, sel_v, sel_o, route_scale):
    s = float(P) ** -0.25
    xf = x[0]
    pad = jnp.zeros((D, HEp - H * E), jnp.float32)
    wqk = jnp.concatenate([Wq.T * s, Wk.T * s], axis=1).astype(jnp.bfloat16)
    v2e = v.astype(jnp.bfloat16).reshape(H, E, D, P).transpose(2, 1, 0, 3)
    v2e = v2e.reshape(D, EHP)
    selw = jnp.concatenate([sel_v.T, pad, sel_o.T, pad], axis=1)

    r = jnp.arange(HEp)[:, None]
    c = jnp.arange(EHP)[None, :]
    rep = (((r % 8) == (c // HP)) & ((r // 8) == ((c % HP) // P)) & (r < H * E))
    rep = rep.astype(jnp.float32) * route_scale[0]
    rep = rep.astype(jnp.bfloat16)

    q2, k2, vmix2, gate_o = pl.pallas_call(
        _proj_kernel,
        grid=(NBLK,),
        in_specs=[
            pl.BlockSpec((SBLK, D), lambda i: (i, 0)),
            pl.BlockSpec((D, 2 * HP), lambda i: (0, 0)),
            pl.BlockSpec((D, EHP), lambda i: (0, 0)),
            pl.BlockSpec((D, 2 * HEp), lambda i: (0, 0)),
            pl.BlockSpec((HEp, EHP), lambda i: (0, 0)),
        ],
        out_specs=[
            pl.BlockSpec((SBLK, HP), lambda i: (i, 0)),
            pl.BlockSpec((SBLK, HP), lambda i: (i, 0)),
            pl.BlockSpec((SBLK, HP), lambda i: (i, 0)),
            pl.BlockSpec((SBLK, HEp), lambda i: (i, 0)),
        ],
        out_shape=[
            jax.ShapeDtypeStruct((S, HP), jnp.bfloat16),
            jax.ShapeDtypeStruct((S, HP), jnp.bfloat16),
            jax.ShapeDtypeStruct((S, HP), jnp.bfloat16),
            jax.ShapeDtypeStruct((S, HEp), jnp.float32),
        ],
    )(xf, wqk, v2e, selw, rep)

    half = S // 2
    r1 = pl.pallas_call(
        functools.partial(_attn_kernel),
        grid=(NBLK // 2,),
        in_specs=[
            pl.BlockSpec((SBLK, HP), lambda i: (i, 0)),
            pl.BlockSpec((half, HP), lambda i: (0, 0)),
            pl.BlockSpec((half, HP), lambda i: (0, 0)),
        ],
        out_specs=pl.BlockSpec((SBLK, HP), lambda i: (i, 0)),
        out_shape=jax.ShapeDtypeStruct((half, HP), jnp.float32),
    )(q2, k2, vmix2)

    r2 = pl.pallas_call(
        functools.partial(_attn_kernel, q_off=NBLK // 2),
        grid=(NBLK // 2,),
        in_specs=[
            pl.BlockSpec((SBLK, HP), lambda i: (i + NBLK // 2, 0)),
            pl.BlockSpec((S, HP), lambda i: (0, 0)),
            pl.BlockSpec((S, HP), lambda i: (0, 0)),
        ],
        out_specs=pl.BlockSpec((SBLK, HP), lambda i: (i, 0)),
        out_shape=jax.ShapeDtypeStruct((half, HP), jnp.float32),
    )(q2, k2, vmix2)

    res2 = jnp.concatenate([r1, r2], axis=0)

    o2e = o.astype(jnp.bfloat16).reshape(H, E, P, D).transpose(1, 0, 2, 3)
    o2e = o2e.reshape(EHP, D)

    out = pl.pallas_call(
        _out_kernel,
        grid=(NBLK,),
        in_specs=[
            pl.BlockSpec((SBLK, HP), lambda i: (i, 0)),
            pl.BlockSpec((SBLK, HEp), lambda i: (i, 0)),
            pl.BlockSpec((HEp, EHP), lambda i: (0, 0)),
            pl.BlockSpec((EHP, D), lambda i: (0, 0)),
        ],
        out_specs=pl.BlockSpec((SBLK, D), lambda i: (i, 0)),
        out_shape=jax.ShapeDtypeStruct((S, D), jnp.float32),
    )(res2, gate_o, rep, o2e)

    return out.reshape(B, S, D)
